# baseline (speedup 1.0000x reference)
import functools
import math

import jax
import jax.numpy as jnp
from jax import lax
from jax.experimental import pallas as pl
from jax.experimental.pallas import tpu as pltpu
from jax.experimental.pallas import tpu_sc as plsc

N_NODES = 10000
N_EDGES = 320000
HID = 128
N_ITERS = 3
N_GRAPHS = 64
EPS = 1e-5

NW = 32
NB = 128
EP = 331776
NBLK = EP // (NW * NB)
ROWS_W = EP // NW
CH_AL = 2048
CH_MLP = 2000
NP_PAD = 10240
SB = 3
SBR = SB * NB
NSB = NBLK // SB
XW = 144

f32 = jnp.float32
i32 = jnp.int32


def _mesh():
    return plsc.VectorSubcoreMesh(core_axis_name="c", subcore_axis_name="s")


@functools.lru_cache(maxsize=None)
def _sc_gather2():
    @functools.partial(
        pl.kernel,
        out_type=jax.ShapeDtypeStruct((EP, HID), f32),
        mesh=_mesh(),
        scratch_types=[
            pltpu.VMEM((ROWS_W,), i32),
            pltpu.VMEM((ROWS_W,), i32),
            pltpu.VMEM((SBR, HID), f32),
            pltpu.VMEM((SBR, HID), f32),
            pltpu.SemaphoreType.DMA,
            pltpu.SemaphoreType.DMA,
        ],
    )
    def k(ta, tb, ia, ib, out, ia_v, ib_v, ra, rb, s1, s2):
        wid = lax.axis_index("s") * 2 + lax.axis_index("c")
        base = wid * ROWS_W
        pltpu.sync_copy(ia.at[pl.ds(base, ROWS_W)], ia_v)
        pltpu.sync_copy(ib.at[pl.ds(base, ROWS_W)], ib_v)

        def sb(b, _):
            off = b * SBR
            descs = []
            for j in range(SB):
                sl = pl.ds(j * NB, NB)
                isl = pl.ds(off + j * NB, NB)
                descs.append(pltpu.async_copy(ta.at[ia_v.at[isl]], ra.at[sl], s1))
                descs.append(pltpu.async_copy(tb.at[ib_v.at[isl]], rb.at[sl], s2))
            for d in descs:
                d.wait()

            def addrow(j, _):
                for kk in range(HID // 16):
                    sl2 = pl.ds(kk * 16, 16)
                    ra[j, sl2] = ra[j, sl2] + rb[j, sl2]
                return 0

            lax.fori_loop(0, SBR, addrow, 0, unroll=4)
            pltpu.sync_copy(ra, out.at[pl.ds(base + off, SBR)])
            return 0

        lax.fori_loop(0, NSB, sb, 0)

    return k


@functools.lru_cache(maxsize=None)
def _sc_agg():
    @functools.partial(
        pl.kernel,
        out_type=jax.ShapeDtypeStruct((2, NP_PAD, HID), f32),
        mesh=_mesh(),
        scratch_types=[
            pltpu.VMEM((ROWS_W,), i32),
            pltpu.VMEM((ROWS_W,), f32),
            pltpu.VMEM((NB,), i32),
            pltpu.VMEM((NB, HID), f32),
            pltpu.VMEM((16,), f32),
            pltpu.VMEM((16, HID), f32),
            pltpu.VMEM_SHARED((NP_PAD, HID), f32),
            pltpu.SemaphoreType.DMA,
        ],
    )
    def k(tbl, ia, ib, al, gm, out, ia_v, al_v, ib0, rv, gv, zv, acc, s1):
        cid = lax.axis_index("c")
        tid = lax.axis_index("s")
        wid = tid * 2 + cid
        base = wid * ROWS_W
        zero = jnp.zeros((16,), f32)

        def zrow(j, _):
            for kk in range(HID // 16):
                zv[j, pl.ds(kk * 16, 16)] = zero
            return 0

        lax.fori_loop(0, 16, zrow, 0)
        for t in range(40):
            pltpu.sync_copy(zv, acc.at[pl.ds(tid * 640 + t * 16, 16)])
        plsc.subcore_barrier()
        pltpu.sync_copy(gm, gv)
        pltpu.sync_copy(ia.at[pl.ds(base, ROWS_W)], ia_v)
        pltpu.sync_copy(al.at[pl.ds(base, ROWS_W)], al_v)
        gmv0 = gv[...]

        def blk(b, _):
            off = b * NB
            pltpu.sync_copy(ib.at[pl.ds(base + off, NB)], ib0)
            pltpu.async_copy(
                tbl.at[ia_v.at[pl.ds(off, NB)]], rv, s1
            ).wait()

            def grp(g, _):
                a16 = al_v[pl.ds(off + g * 16, 16)]
                aexp = jnp.exp(a16 - gmv0)
                for j in range(16):
                    sp = jnp.broadcast_to(aexp[j], (16,))
                    for kk in range(HID // 16):
                        sl2 = pl.ds(kk * 16, 16)
                        rv[g * 16 + j, sl2] = rv[g * 16 + j, sl2] * sp
                return 0

            lax.fori_loop(0, NB // 16, grp, 0)
            pltpu.sync_copy(rv, acc.at[ib0], add=True)
            return 0

        lax.fori_loop(0, NBLK, blk, 0)
        plsc.subcore_barrier()
        pltpu.sync_copy(
            acc.at[pl.ds(tid * 640, 640)], out.at[cid, pl.ds(tid * 640, 640)]
        )

    return k


@functools.lru_cache(maxsize=None)
def _sc_segsum(do_exp=False):
    scratch = [
        pltpu.VMEM((NB,), f32),
        pltpu.VMEM((NB,), i32),
        pltpu.VMEM((NP_PAD,), f32),
        pltpu.VMEM((640,), f32),
        pltpu.VMEM((640,), f32),
        pltpu.VMEM((16,), f32),
        pltpu.VMEM_SHARED((16 * NP_PAD,), f32),
    ]

    @functools.partial(
        pl.kernel,
        out_type=jax.ShapeDtypeStruct((2 * NP_PAD,), f32),
        mesh=_mesh(),
        scratch_types=scratch,
    )
    def k(ev, dv, gm, out, ev_v, dv_v, part, accv, tmpv, gv, sh):
        cid = lax.axis_index("c")
        tid = lax.axis_index("s")
        wid = tid * 2 + cid
        base = wid * ROWS_W
        zero = jnp.zeros((16,), f32)
        lane0 = lax.iota(i32, 16) == 0

        def z1(i, _):
            part[pl.ds(i * 16, 16)] = zero
            return 0

        lax.fori_loop(0, NP_PAD // 16, z1, 0)
        pltpu.sync_copy(gm, gv)
        gmv = gv[...]

        def blk(b, _):
            off = base + b * NB
            pltpu.sync_copy(ev.at[pl.ds(off, NB)], ev_v)
            pltpu.sync_copy(dv.at[pl.ds(off, NB)], dv_v)

            def grp(g, _):
                sl = pl.ds(g * 16, 16)
                v16 = ev_v[sl]
                if do_exp:
                    v16 = jnp.exp(v16 - gmv)
                d16 = dv_v[sl]
                for j in range(16):
                    dsl = pl.ds(d16[j], 16)
                    contrib = jnp.where(
                        lane0, jnp.broadcast_to(v16[j], (16,)), 0.0
                    )
                    part[dsl] = part[dsl] + contrib
                return 0

            lax.fori_loop(0, NB // 16, grp, 0)
            return 0

        lax.fori_loop(0, NBLK, blk, 0)
        pltpu.sync_copy(part, sh.at[pl.ds(tid * NP_PAD, NP_PAD)])
        plsc.subcore_barrier()
        base_o = tid * 640

        def z2(i, _):
            accv[pl.ds(i * 16, 16)] = zero
            return 0

        lax.fori_loop(0, 40, z2, 0)
        for t in range(16):
            pltpu.sync_copy(sh.at[pl.ds(t * NP_PAD + base_o, 640)], tmpv)

            def addv(i, _):
                sl = pl.ds(i * 16, 16)
                accv[sl] = accv[sl] + tmpv[sl]
                return 0

            lax.fori_loop(0, 40, addv, 0)
        pltpu.sync_copy(accv, out.at[pl.ds(cid * NP_PAD + base_o, 640)])

    return k


_ZERO16 = None


def _segsum(vals_pad, dstp, gmax16=None):
    do_exp = gmax16 is not None
    if gmax16 is None:
        gmax16 = jnp.zeros((16,), f32)
    out = _sc_segsum(do_exp)(vals_pad, dstp, gmax16)
    return out[:N_NODES] + out[NP_PAD : NP_PAD + N_NODES]


def _alpha_body(u_ref, a_ref, we_ref, att_ref, al_ref, gm_ref):
    i = pl.program_id(0)
    m = u_ref[...] + a_ref[...] * we_ref[...]
    m = jax.nn.leaky_relu(m, 0.2)
    ab = jnp.dot(
        m.astype(jnp.bfloat16), att_ref[...], preferred_element_type=f32
    )[:, :1]
    rid = i * CH_AL + lax.broadcasted_iota(i32, (CH_AL, 1), 0)
    ab = jnp.where(rid < N_EDGES + N_NODES, ab, -3e38)
    al_ref[...] = ab
    bm = jnp.max(ab).reshape(1, 1)

    @pl.when(i == 0)
    def _():
        gm_ref[...] = bm

    @pl.when(i != 0)
    def _():
        gm_ref[...] = jnp.maximum(gm_ref[...], bm)


def _alpha(U, a_f_pad, w_e, att):
    grid = EP // CH_AL
    attp = jnp.zeros((HID, 128), jnp.bfloat16).at[:, 0].set(att.astype(jnp.bfloat16))
    return pl.pallas_call(
        _alpha_body,
        grid=(grid,),
        in_specs=[
            pl.BlockSpec((CH_AL, HID), lambda i: (i, 0)),
            pl.BlockSpec((CH_AL, 1), lambda i: (i, 0)),
            pl.BlockSpec((1, HID), lambda i: (0, 0)),
            pl.BlockSpec((HID, 128), lambda i: (0, 0)),
        ],
        out_specs=[
            pl.BlockSpec((CH_AL, 1), lambda i: (i, 0)),
            pl.BlockSpec((1, 1), lambda i: (0, 0)),
        ],
        out_shape=[
            jax.ShapeDtypeStruct((EP, 1), f32),
            jax.ShapeDtypeStruct((1, 1), f32),
        ],
    )(U, a_f_pad[:, None], w_e[None, :], attp)


def _mlp_body(u_ref, e_ref, w1c_ref, b1_ref, lnw_ref, lnb_ref, w2_ref, b2_ref,
              w3_ref, b3_ref, w4_ref, b4_ref, out_ref):
    u = u_ref[...] + e_ref[...].astype(f32) * w1c_ref[...].astype(f32) + b1_ref[0:1]

    def ln_tanh(z, j):
        mu = jnp.mean(z, axis=-1, keepdims=True)
        var = jnp.mean((z - mu) ** 2, axis=-1, keepdims=True)
        z = (z - mu) / jnp.sqrt(var + EPS) * lnw_ref[j : j + 1] + lnb_ref[j : j + 1]
        return jnp.tanh(z)

    u = ln_tanh(u, 0)
    u = jnp.dot(u.astype(jnp.bfloat16), w2_ref[...], preferred_element_type=f32)
    u = ln_tanh(u + b2_ref[0:1], 1)
    u = jnp.dot(u.astype(jnp.bfloat16), w3_ref[...], preferred_element_type=f32)
    u = ln_tanh(u + b3_ref[0:1], 2)
    u = jnp.dot(u.astype(jnp.bfloat16), w4_ref[...], preferred_element_type=f32)
    out_ref[...] = jax.nn.sigmoid(u[:, :1] + b4_ref[0, 0])


def _edge_mlp(U2, e, emlp):
    grid = N_EDGES // CH_MLP
    w1 = emlp["lins"][0]["w"]
    w1c = w1[:, 2 * HID].astype(jnp.bfloat16)
    b1 = emlp["lins"][0]["b"]
    lnw = jnp.stack([ln["w"] for ln in emlp["lns"]])
    lnb = jnp.stack([ln["b"] for ln in emlp["lns"]])
    w2 = emlp["lins"][1]["w"].T.astype(jnp.bfloat16)
    b2 = emlp["lins"][1]["b"]
    w3 = emlp["lins"][2]["w"].T.astype(jnp.bfloat16)
    b3 = emlp["lins"][2]["b"]
    w4 = jnp.zeros((HID, 128), jnp.bfloat16).at[:, 0].set(
        emlp["lins"][3]["w"][0].astype(jnp.bfloat16)
    )
    b4 = emlp["lins"][3]["b"].reshape(1, 1)
    bspec = lambda r, c: pl.BlockSpec((r, c), lambda i: (0, 0))
    out = pl.pallas_call(
        _mlp_body,
        grid=(grid,),
        in_specs=[
            pl.BlockSpec((CH_MLP, HID), lambda i: (i, 0)),
            pl.BlockSpec((CH_MLP, 1), lambda i: (i, 0)),
            bspec(1, HID), bspec(1, HID), bspec(3, HID), bspec(3, HID),
            bspec(HID, HID), bspec(1, HID), bspec(HID, HID), bspec(1, HID),
            bspec(HID, 128), bspec(1, 1),
        ],
        out_specs=pl.BlockSpec((CH_MLP, 1), lambda i: (i, 0)),
        out_shape=jax.ShapeDtypeStruct((N_EDGES, 1), f32),
    )(U2, e.astype(jnp.bfloat16)[:, None], w1c[None, :], b1[None, :],
      lnw, lnb, w2, b2[None, :], w3, b3[None, :], w4, b4)
    return out[:, 0]


_POOL_CHUNK = 1024


def _pool_body(batch_ref, h_ref, out_ref):
    i = pl.program_id(0)
    b = batch_ref[...].astype(i32)
    gid = lax.broadcasted_iota(i32, (N_GRAPHS, _POOL_CHUNK), 0)
    onehot = (b[None, :] == gid).astype(f32)
    part = jnp.dot(onehot, h_ref[...], preferred_element_type=f32,
                   precision=lax.Precision.HIGHEST)

    @pl.when(i == 0)
    def _init():
        out_ref[...] = part

    @pl.when(i != 0)
    def _acc():
        out_ref[...] += part


def _pool(h, batch):
    n_pad = math.ceil(N_NODES / _POOL_CHUNK) * _POOL_CHUNK
    hp = jnp.pad(h, ((0, n_pad - N_NODES), (0, 0)))
    bp = jnp.pad(batch, (0, n_pad - N_NODES), constant_values=N_GRAPHS)
    grid = n_pad // _POOL_CHUNK
    return pl.pallas_call(
        _pool_body,
        grid=(grid,),
        in_specs=[
            pl.BlockSpec((_POOL_CHUNK,), lambda i: (i,)),
            pl.BlockSpec((_POOL_CHUNK, HID), lambda i: (i, 0)),
        ],
        out_specs=pl.BlockSpec((N_GRAPHS, HID), lambda i: (0, 0)),
        out_shape=jax.ShapeDtypeStruct((N_GRAPHS, HID), f32),
    )(bp, hp)


def _layernorm(z, w, b):
    mu = jnp.mean(z, axis=-1, keepdims=True)
    var = jnp.var(z, axis=-1, keepdims=True)
    return (z - mu) / jnp.sqrt(var + EPS) * w + b


def _graph_ln(z, w, b):
    z = z - jnp.mean(z)
    return z / (jnp.std(z) + EPS) * w + b


def _mlp_tail(z, ps):
    for p, ln in zip(ps["lins"][:-1], ps["lns"]):
        z = z @ p["w"].T + p["b"]
        z = _layernorm(z, ln["w"], ln["b"])
        z = jnp.tanh(z)
    p = ps["lins"][-1]
    return z @ p["w"].T + p["b"]


def _bdot(a, b):
    return jnp.dot(a.astype(jnp.bfloat16), b.astype(jnp.bfloat16),
                   preferred_element_type=f32)


def _gat_layer(z, srcf_p, dstf_p, a_f_pad, g):
    x_l = _bdot(z, g["lin_l"]["w"].T) + g["lin_l"]["b"]
    x_r = _bdot(z, g["lin_r"]["w"].T) + g["lin_r"]["b"]
    w_e = g["lin_edge"]["w"][:, 0]
    U = _sc_gather2()(x_l, x_r, srcf_p, dstf_p)
    alpha, gmax = _alpha(U, a_f_pad, w_e, g["att"])
    gmax16 = jnp.broadcast_to(gmax[0, 0], (16,))
    al = alpha[:, 0]
    asum = _segsum(al, dstf_p, gmax16)
    acc2 = _sc_agg()(x_l, srcf_p, dstf_p, al, gmax16)
    acc = acc2[0, :N_NODES] + acc2[1, :N_NODES]
    out = acc / (asum[:, None] + 1e-16)
    return out + g["bias"]


def kernel(x, e, params, edge_index, batch):
    src, dst = edge_index[0], edge_index[1]
    n = N_NODES
    ar = jnp.arange(n, dtype=src.dtype)
    zpad_f = jnp.zeros((EP - N_EDGES - n,), i32)
    zpad_e = jnp.zeros((EP - N_EDGES,), i32)
    srcf_p = jnp.concatenate([src, ar, zpad_f])
    dstf_p = jnp.concatenate([dst, ar, zpad_f])
    src_p = jnp.concatenate([src, zpad_e])
    dst_p = jnp.concatenate([dst, zpad_e])

    zpad_v = jnp.zeros((EP - N_EDGES,), f32)
    deg = _segsum(jnp.concatenate([jnp.ones((N_EDGES,), f32), zpad_v]), dst_p)
    max_deg = jnp.maximum(deg, 1.0)

    h = x @ params["in_lin"]["w"].T + params["in_lin"]["b"]
    h = _layernorm(h, params["in_ln"]["w"], params["in_ln"]["b"])
    h = jnp.tanh(h)

    emlp = params["edge_mlp"]
    w1 = emlp["lins"][0]["w"]
    w1a, w1b = w1[:, :HID], w1[:, HID : 2 * HID]
    b1 = emlp["lins"][0]["b"]

    for _ in range(N_ITERS):
        x0 = h
        loop_attr = _segsum(jnp.concatenate([e, zpad_v]), dst_p) / max_deg
        a_f_pad = jnp.concatenate([e, loop_attr, jnp.zeros((EP - N_EDGES - n,), f32)])
        z = h
        for li in range(4):
            z = _gat_layer(z, srcf_p, dstf_p, a_f_pad, params["gats"][li])
            if li < 3:
                z = jnp.tanh(z)
                z = _graph_ln(z, params["gat_lns"][li]["w"], params["gat_lns"][li]["b"])
        h = z
        ha = _bdot(h, w1a.T)
        hb = _bdot(h, w1b.T)
        U2 = _sc_gather2()(ha, hb, src_p, dst_p)
        e = _edge_mlp(U2, e, emlp)
        h = h + x0

    s = _pool(h, batch)
    return _mlp_tail(s, params["pred_mlp"])[:, 0]


# speedup vs baseline: 1.1655x; 1.1655x over previous
import functools
import math

import jax
import jax.numpy as jnp
from jax import lax
from jax.experimental import pallas as pl
from jax.experimental.pallas import tpu as pltpu
from jax.experimental.pallas import tpu_sc as plsc

N_NODES = 10000
N_EDGES = 320000
HID = 128
N_ITERS = 3
N_GRAPHS = 64
EPS = 1e-5

NW = 32
NB = 128
EP = 331776
NBLK = EP // (NW * NB)
ROWS_W = EP // NW
CH_AL = 2048
CH_MLP = 2000
NP_PAD = 10240
SB = 3
SBR = SB * NB
NSB = NBLK // SB
XW = 144

f32 = jnp.float32
i32 = jnp.int32


def _mesh():
    return plsc.VectorSubcoreMesh(core_axis_name="c", subcore_axis_name="s")


@functools.lru_cache(maxsize=None)
def _sc_gather2():
    @functools.partial(
        pl.kernel,
        out_type=jax.ShapeDtypeStruct((EP, HID), f32),
        mesh=_mesh(),
        scratch_types=[
            pltpu.VMEM((SBR,), i32),
            pltpu.VMEM((SBR,), i32),
            pltpu.VMEM((SBR, HID), f32),
            pltpu.VMEM((SBR, HID), f32),
            pltpu.SemaphoreType.DMA,
            pltpu.SemaphoreType.DMA,
        ],
    )
    def k(ta, tb, ia, ib, out, ia_v, ib_v, ra, rb, s1, s2):
        wid = lax.axis_index("s") * 2 + lax.axis_index("c")
        base = wid * ROWS_W

        def sb(b, _):
            off = base + b * SBR
            pltpu.sync_copy(ia.at[pl.ds(off, SBR)], ia_v)
            pltpu.sync_copy(ib.at[pl.ds(off, SBR)], ib_v)
            descs = []
            for j in range(SB):
                sl = pl.ds(j * NB, NB)
                descs.append(pltpu.async_copy(ta.at[ia_v.at[sl]], ra.at[sl], s1))
                descs.append(pltpu.async_copy(tb.at[ib_v.at[sl]], rb.at[sl], s2))
            for d in descs:
                d.wait()

            def addrow(j, _):
                for kk in range(HID // 16):
                    sl2 = pl.ds(kk * 16, 16)
                    ra[j, sl2] = ra[j, sl2] + rb[j, sl2]
                return 0

            lax.fori_loop(0, SBR, addrow, 0)
            pltpu.sync_copy(ra, out.at[pl.ds(off, SBR)])
            return 0

        lax.fori_loop(0, NSB, sb, 0)

    return k


@functools.lru_cache(maxsize=None)
def _sc_agg():
    @functools.partial(
        pl.kernel,
        out_type=jax.ShapeDtypeStruct((2, NP_PAD, HID), f32),
        mesh=_mesh(),
        scratch_types=[
            pltpu.VMEM((2 * NB,), i32),
            pltpu.VMEM((NB,), i32),
            pltpu.VMEM((NB,), i32),
            pltpu.VMEM((2 * NB,), f32),
            pltpu.VMEM((2 * NB, HID), f32),
            pltpu.VMEM((16,), f32),
            pltpu.VMEM((32, HID), f32),
            pltpu.VMEM_SHARED((NP_PAD, HID), f32),
            pltpu.SemaphoreType.DMA,
        ],
    )
    def k(tbl, ia, ib, al, gm, out, ia_v, ib0, ib1, av, rv, gv, zv, acc, s1):
        cid = lax.axis_index("c")
        tid = lax.axis_index("s")
        wid = tid * 2 + cid
        base = wid * ROWS_W
        zero = jnp.zeros((16,), f32)

        def zrow(j, _):
            for kk in range(HID // 16):
                zv[j, pl.ds(kk * 16, 16)] = zero
            return 0

        lax.fori_loop(0, 32, zrow, 0)
        for t in range(20):
            pltpu.sync_copy(zv, acc.at[pl.ds(tid * 640 + t * 32, 32)])
        plsc.subcore_barrier()
        pltpu.sync_copy(gm, gv)
        gmv0 = gv[...]

        def scale(nrows):
            def grp(g, _):
                a16 = av[pl.ds(g * 16, 16)]
                aexp = jnp.exp(a16 - gmv0)
                for j in range(16):
                    sp = jnp.broadcast_to(aexp[j], (16,))
                    for kk in range(HID // 16):
                        sl2 = pl.ds(kk * 16, 16)
                        rv[g * 16 + j, sl2] = rv[g * 16 + j, sl2] * sp
                return 0

            lax.fori_loop(0, nrows // 16, grp, 0)

        def sb(b, _):
            off = base + b * 2 * NB
            pltpu.sync_copy(ia.at[pl.ds(off, 2 * NB)], ia_v)
            pltpu.sync_copy(ib.at[pl.ds(off, NB)], ib0)
            pltpu.sync_copy(ib.at[pl.ds(off + NB, NB)], ib1)
            pltpu.sync_copy(al.at[pl.ds(off, 2 * NB)], av)
            sl0 = pl.ds(0, NB)
            sl1 = pl.ds(NB, NB)
            d0 = pltpu.async_copy(tbl.at[ia_v.at[sl0]], rv.at[sl0], s1)
            d1 = pltpu.async_copy(tbl.at[ia_v.at[sl1]], rv.at[sl1], s1)
            d0.wait()
            d1.wait()
            scale(2 * NB)
            pltpu.sync_copy(rv.at[sl0], acc.at[ib0], add=True)
            pltpu.sync_copy(rv.at[sl1], acc.at[ib1], add=True)
            return 0

        lax.fori_loop(0, NBLK // 2, sb, 0)
        off = base + (NBLK - 1) * NB
        pltpu.sync_copy(ia.at[pl.ds(off, NB)], ia_v.at[pl.ds(0, NB)])
        pltpu.sync_copy(ib.at[pl.ds(off, NB)], ib0)
        pltpu.sync_copy(al.at[pl.ds(off, NB)], av.at[pl.ds(0, NB)])
        pltpu.async_copy(tbl.at[ia_v.at[pl.ds(0, NB)]], rv.at[pl.ds(0, NB)], s1).wait()
        scale(NB)
        pltpu.sync_copy(rv.at[pl.ds(0, NB)], acc.at[ib0], add=True)
        plsc.subcore_barrier()
        pltpu.sync_copy(
            acc.at[pl.ds(tid * 640, 640)], out.at[cid, pl.ds(tid * 640, 640)]
        )

    return k


@functools.lru_cache(maxsize=None)
def _sc_segsum(do_exp=False):
    scratch = [
        pltpu.VMEM((NB,), f32),
        pltpu.VMEM((NB,), i32),
        pltpu.VMEM((NP_PAD,), f32),
        pltpu.VMEM((640,), f32),
        pltpu.VMEM((640,), f32),
        pltpu.VMEM((16,), f32),
        pltpu.VMEM_SHARED((16 * NP_PAD,), f32),
    ]

    @functools.partial(
        pl.kernel,
        out_type=jax.ShapeDtypeStruct((2 * NP_PAD,), f32),
        mesh=_mesh(),
        scratch_types=scratch,
    )
    def k(ev, dv, gm, out, ev_v, dv_v, part, accv, tmpv, gv, sh):
        cid = lax.axis_index("c")
        tid = lax.axis_index("s")
        wid = tid * 2 + cid
        base = wid * ROWS_W
        zero = jnp.zeros((16,), f32)
        lane0 = lax.iota(i32, 16) == 0

        def z1(i, _):
            part[pl.ds(i * 16, 16)] = zero
            return 0

        lax.fori_loop(0, NP_PAD // 16, z1, 0)
        pltpu.sync_copy(gm, gv)
        gmv = gv[...]

        def blk(b, _):
            off = base + b * NB
            pltpu.sync_copy(ev.at[pl.ds(off, NB)], ev_v)
            pltpu.sync_copy(dv.at[pl.ds(off, NB)], dv_v)

            def grp(g, _):
                sl = pl.ds(g * 16, 16)
                v16 = ev_v[sl]
                if do_exp:
                    v16 = jnp.exp(v16 - gmv)
                d16 = dv_v[sl]
                for j in range(16):
                    dsl = pl.ds(d16[j], 16)
                    contrib = jnp.where(
                        lane0, jnp.broadcast_to(v16[j], (16,)), 0.0
                    )
                    part[dsl] = part[dsl] + contrib
                return 0

            lax.fori_loop(0, NB // 16, grp, 0)
            return 0

        lax.fori_loop(0, NBLK, blk, 0)
        pltpu.sync_copy(part, sh.at[pl.ds(tid * NP_PAD, NP_PAD)])
        plsc.subcore_barrier()
        base_o = tid * 640

        def z2(i, _):
            accv[pl.ds(i * 16, 16)] = zero
            return 0

        lax.fori_loop(0, 40, z2, 0)
        for t in range(16):
            pltpu.sync_copy(sh.at[pl.ds(t * NP_PAD + base_o, 640)], tmpv)

            def addv(i, _):
                sl = pl.ds(i * 16, 16)
                accv[sl] = accv[sl] + tmpv[sl]
                return 0

            lax.fori_loop(0, 40, addv, 0)
        pltpu.sync_copy(accv, out.at[pl.ds(cid * NP_PAD + base_o, 640)])

    return k


_ZERO16 = None


def _segsum(vals_pad, dstp, gmax16=None):
    do_exp = gmax16 is not None
    if gmax16 is None:
        gmax16 = jnp.zeros((16,), f32)
    out = _sc_segsum(do_exp)(vals_pad, dstp, gmax16)
    return out[:N_NODES] + out[NP_PAD : NP_PAD + N_NODES]


def _alpha_body(u_ref, a_ref, we_ref, att_ref, al_ref, gm_ref):
    i = pl.program_id(0)
    m = u_ref[...] + a_ref[...] * we_ref[...]
    m = jax.nn.leaky_relu(m, 0.2)
    ab = jnp.dot(
        m.astype(jnp.bfloat16), att_ref[...], preferred_element_type=f32
    )[:, :1]
    rid = i * CH_AL + lax.broadcasted_iota(i32, (CH_AL, 1), 0)
    ab = jnp.where(rid < N_EDGES + N_NODES, ab, -3e38)
    al_ref[...] = ab
    bm = jnp.max(ab).reshape(1, 1)

    @pl.when(i == 0)
    def _():
        gm_ref[...] = bm

    @pl.when(i != 0)
    def _():
        gm_ref[...] = jnp.maximum(gm_ref[...], bm)


def _alpha(U, a_f_pad, w_e, att):
    grid = EP // CH_AL
    attp = jnp.zeros((HID, 128), jnp.bfloat16).at[:, 0].set(att.astype(jnp.bfloat16))
    return pl.pallas_call(
        _alpha_body,
        grid=(grid,),
        in_specs=[
            pl.BlockSpec((CH_AL, HID), lambda i: (i, 0)),
            pl.BlockSpec((CH_AL, 1), lambda i: (i, 0)),
            pl.BlockSpec((1, HID), lambda i: (0, 0)),
            pl.BlockSpec((HID, 128), lambda i: (0, 0)),
        ],
        out_specs=[
            pl.BlockSpec((CH_AL, 1), lambda i: (i, 0)),
            pl.BlockSpec((1, 1), lambda i: (0, 0)),
        ],
        out_shape=[
            jax.ShapeDtypeStruct((EP, 1), f32),
            jax.ShapeDtypeStruct((1, 1), f32),
        ],
    )(U, a_f_pad[:, None], w_e[None, :], attp)


def _mlp_body(u_ref, e_ref, w1c_ref, b1_ref, lnw_ref, lnb_ref, w2_ref, b2_ref,
              w3_ref, b3_ref, w4_ref, b4_ref, out_ref):
    u = u_ref[...] + e_ref[...].astype(f32) * w1c_ref[...].astype(f32) + b1_ref[0:1]

    def ln_tanh(z, j):
        mu = jnp.mean(z, axis=-1, keepdims=True)
        var = jnp.mean((z - mu) ** 2, axis=-1, keepdims=True)
        z = (z - mu) / jnp.sqrt(var + EPS) * lnw_ref[j : j + 1] + lnb_ref[j : j + 1]
        return jnp.tanh(z)

    u = ln_tanh(u, 0)
    u = jnp.dot(u.astype(jnp.bfloat16), w2_ref[...], preferred_element_type=f32)
    u = ln_tanh(u + b2_ref[0:1], 1)
    u = jnp.dot(u.astype(jnp.bfloat16), w3_ref[...], preferred_element_type=f32)
    u = ln_tanh(u + b3_ref[0:1], 2)
    u = jnp.dot(u.astype(jnp.bfloat16), w4_ref[...], preferred_element_type=f32)
    out_ref[...] = jax.nn.sigmoid(u[:, :1] + b4_ref[0, 0])


def _edge_mlp(U2, e, emlp):
    grid = N_EDGES // CH_MLP
    w1 = emlp["lins"][0]["w"]
    w1c = w1[:, 2 * HID].astype(jnp.bfloat16)
    b1 = emlp["lins"][0]["b"]
    lnw = jnp.stack([ln["w"] for ln in emlp["lns"]])
    lnb = jnp.stack([ln["b"] for ln in emlp["lns"]])
    w2 = emlp["lins"][1]["w"].T.astype(jnp.bfloat16)
    b2 = emlp["lins"][1]["b"]
    w3 = emlp["lins"][2]["w"].T.astype(jnp.bfloat16)
    b3 = emlp["lins"][2]["b"]
    w4 = jnp.zeros((HID, 128), jnp.bfloat16).at[:, 0].set(
        emlp["lins"][3]["w"][0].astype(jnp.bfloat16)
    )
    b4 = emlp["lins"][3]["b"].reshape(1, 1)
    bspec = lambda r, c: pl.BlockSpec((r, c), lambda i: (0, 0))
    out = pl.pallas_call(
        _mlp_body,
        grid=(grid,),
        in_specs=[
            pl.BlockSpec((CH_MLP, HID), lambda i: (i, 0)),
            pl.BlockSpec((CH_MLP, 1), lambda i: (i, 0)),
            bspec(1, HID), bspec(1, HID), bspec(3, HID), bspec(3, HID),
            bspec(HID, HID), bspec(1, HID), bspec(HID, HID), bspec(1, HID),
            bspec(HID, 128), bspec(1, 1),
        ],
        out_specs=pl.BlockSpec((CH_MLP, 1), lambda i: (i, 0)),
        out_shape=jax.ShapeDtypeStruct((N_EDGES, 1), f32),
    )(U2, e.astype(jnp.bfloat16)[:, None], w1c[None, :], b1[None, :],
      lnw, lnb, w2, b2[None, :], w3, b3[None, :], w4, b4)
    return out[:, 0]


_POOL_CHUNK = 1024


def _pool_body(batch_ref, h_ref, out_ref):
    i = pl.program_id(0)
    b = batch_ref[...].astype(i32)
    gid = lax.broadcasted_iota(i32, (N_GRAPHS, _POOL_CHUNK), 0)
    onehot = (b[None, :] == gid).astype(f32)
    part = jnp.dot(onehot, h_ref[...], preferred_element_type=f32,
                   precision=lax.Precision.HIGHEST)

    @pl.when(i == 0)
    def _init():
        out_ref[...] = part

    @pl.when(i != 0)
    def _acc():
        out_ref[...] += part


def _pool(h, batch):
    n_pad = math.ceil(N_NODES / _POOL_CHUNK) * _POOL_CHUNK
    hp = jnp.pad(h, ((0, n_pad - N_NODES), (0, 0)))
    bp = jnp.pad(batch, (0, n_pad - N_NODES), constant_values=N_GRAPHS)
    grid = n_pad // _POOL_CHUNK
    return pl.pallas_call(
        _pool_body,
        grid=(grid,),
        in_specs=[
            pl.BlockSpec((_POOL_CHUNK,), lambda i: (i,)),
            pl.BlockSpec((_POOL_CHUNK, HID), lambda i: (i, 0)),
        ],
        out_specs=pl.BlockSpec((N_GRAPHS, HID), lambda i: (0, 0)),
        out_shape=jax.ShapeDtypeStruct((N_GRAPHS, HID), f32),
    )(bp, hp)


def _layernorm(z, w, b):
    mu = jnp.mean(z, axis=-1, keepdims=True)
    var = jnp.var(z, axis=-1, keepdims=True)
    return (z - mu) / jnp.sqrt(var + EPS) * w + b


def _graph_ln(z, w, b):
    z = z - jnp.mean(z)
    return z / (jnp.std(z) + EPS) * w + b


def _mlp_tail(z, ps):
    for p, ln in zip(ps["lins"][:-1], ps["lns"]):
        z = z @ p["w"].T + p["b"]
        z = _layernorm(z, ln["w"], ln["b"])
        z = jnp.tanh(z)
    p = ps["lins"][-1]
    return z @ p["w"].T + p["b"]


def _bdot(a, b):
    return jnp.dot(a.astype(jnp.bfloat16), b.astype(jnp.bfloat16),
                   preferred_element_type=f32)


def _gat_layer(z, srcf_p, dstf_p, a_f_pad, g):
    x_l = _bdot(z, g["lin_l"]["w"].T) + g["lin_l"]["b"]
    x_r = _bdot(z, g["lin_r"]["w"].T) + g["lin_r"]["b"]
    w_e = g["lin_edge"]["w"][:, 0]
    U = _sc_gather2()(x_l, x_r, srcf_p, dstf_p)
    alpha, gmax = _alpha(U, a_f_pad, w_e, g["att"])
    gmax16 = jnp.broadcast_to(gmax[0, 0], (16,))
    al = alpha[:, 0]
    asum = _segsum(al, dstf_p, gmax16)
    acc2 = _sc_agg()(x_l, srcf_p, dstf_p, al, gmax16)
    acc = acc2[0, :N_NODES] + acc2[1, :N_NODES]
    out = acc / (asum[:, None] + 1e-16)
    return out + g["bias"]


def kernel(x, e, params, edge_index, batch):
    src, dst = edge_index[0], edge_index[1]
    n = N_NODES
    ar = jnp.arange(n, dtype=src.dtype)
    zpad_f = jnp.zeros((EP - N_EDGES - n,), i32)
    zpad_e = jnp.zeros((EP - N_EDGES,), i32)
    srcf_p = jnp.concatenate([src, ar, zpad_f])
    dstf_p = jnp.concatenate([dst, ar, zpad_f])
    src_p = jnp.concatenate([src, zpad_e])
    dst_p = jnp.concatenate([dst, zpad_e])

    zpad_v = jnp.zeros((EP - N_EDGES,), f32)
    deg = _segsum(jnp.concatenate([jnp.ones((N_EDGES,), f32), zpad_v]), dst_p)
    max_deg = jnp.maximum(deg, 1.0)

    h = x @ params["in_lin"]["w"].T + params["in_lin"]["b"]
    h = _layernorm(h, params["in_ln"]["w"], params["in_ln"]["b"])
    h = jnp.tanh(h)

    emlp = params["edge_mlp"]
    w1 = emlp["lins"][0]["w"]
    w1a, w1b = w1[:, :HID], w1[:, HID : 2 * HID]
    b1 = emlp["lins"][0]["b"]

    for _ in range(N_ITERS):
        x0 = h
        loop_attr = _segsum(jnp.concatenate([e, zpad_v]), dst_p) / max_deg
        a_f_pad = jnp.concatenate([e, loop_attr, jnp.zeros((EP - N_EDGES - n,), f32)])
        z = h
        for li in range(4):
            z = _gat_layer(z, srcf_p, dstf_p, a_f_pad, params["gats"][li])
            if li < 3:
                z = jnp.tanh(z)
                z = _graph_ln(z, params["gat_lns"][li]["w"], params["gat_lns"][li]["b"])
        h = z
        ha = _bdot(h, w1a.T)
        hb = _bdot(h, w1b.T)
        U2 = _sc_gather2()(ha, hb, src_p, dst_p)
        e = _edge_mlp(U2, e, emlp)
        h = h + x0

    s = _pool(h, batch)
    return _mlp_tail(s, params["pred_mlp"])[:, 0]
